# baseline (speedup 1.0000x reference)
import functools

import jax
import jax.numpy as jnp
from jax import lax
from jax.experimental import pallas as pl
from jax.experimental.pallas import tpu as pltpu
from jax.experimental.pallas import tpu_sc as plsc


B = 4
S = 2048
KK = 64
CB = 128
RBM = 256
NJG = 8
NC = 2
NSUB = 16


def _topk_body(scores_t_ref, idx_ref):
    vals0 = scores_t_ref[0]
    iota_c = lax.broadcasted_iota(jnp.int32, (S, CB), 0)
    sub8 = lax.broadcasted_iota(jnp.int32, (8, CB), 0)

    def step(t, vals):
        v, ix = vals, iota_c
        while v.shape[0] > 1:
            h = v.shape[0] // 2
            a, b = v[:h], v[h:]
            ia, ib = ix[:h], ix[h:]
            gt = (a > b) | ((a == b) & (ia < ib))
            v = jnp.where(gt, a, b)
            ix = jnp.where(gt, ia, ib)
        am = ix
        idx_ref[0, pl.ds(t, 1), :] = am
        av = am >> 3
        asub = (am & 7) == sub8
        pieces = []
        for vi in range(S // 8):
            hit = (av == vi) & asub
            pieces.append(jnp.where(hit, -jnp.inf,
                                    vals[8 * vi:8 * vi + 8, :]))
        return jnp.concatenate(pieces, axis=0)

    lax.fori_loop(0, KK, step, vals0)


def _sc_scatter_body(idx_t_hbm, pres_hbm, idx_v, pres_v):
    wid = lax.axis_index("s") * NC + lax.axis_index("c")
    b = wid // NJG
    jg = wid % NJG
    pltpu.sync_copy(idx_t_hbm.at[b, pl.ds(jg * 8, 8)], idx_v)

    zeros = jnp.zeros((16,), jnp.int32)
    ones = jnp.ones((16,), jnp.int32)

    def zero_step(i, carry):
        pres_v[pl.ds(i * 16, 16)] = zeros
        return carry

    lax.fori_loop(0, S * 8 // 16, zero_step, 0)

    for jl in range(8):
        jl_vec = jnp.full((16,), jl, jnp.int32)

        def mark_step(i, carry):
            rows = idx_v[jl, pl.ds(i * 16, 16)]
            plsc.store_scatter(pres_v, [rows * 8 + jl_vec], ones)
            return carry

        lax.fori_loop(0, S // 16, mark_step, 0)

    pltpu.sync_copy(pres_v, pres_hbm.at[b, jg])


def _mask_body(p0, p1, p2, p3, p4, p5, p6, p7, out_ref):
    ri = pl.program_id(1)
    pres = jnp.concatenate(
        [p[0, 0] for p in (p0, p1, p2, p3, p4, p5, p6, p7)], axis=1)
    pres_full = jnp.concatenate(
        [pres, jnp.zeros((RBM, S - KK), dtype=jnp.int32)], axis=1)
    keep = pres_full == 0
    rr = lax.broadcasted_iota(jnp.int32, (RBM, S), 0) + ri * RBM
    jj = lax.broadcasted_iota(jnp.int32, (RBM, S), 1)
    override = (rr <= KK) & (jj > rr)
    out_ref[0, 0] = keep | override


def kernel(attn_scores):
    scores_t = jnp.swapaxes(attn_scores, 1, 2)

    idx_t = pl.pallas_call(
        _topk_body,
        grid=(B, S // CB),
        in_specs=[pl.BlockSpec((1, S, CB), lambda b, si: (b, 0, si))],
        out_specs=pl.BlockSpec((1, KK, CB), lambda b, si: (b, 0, si)),
        out_shape=jax.ShapeDtypeStruct((B, KK, S), jnp.int32),
    )(scores_t)

    mesh = plsc.VectorSubcoreMesh(core_axis_name="c", subcore_axis_name="s")
    pres = functools.partial(
        pl.kernel,
        mesh=mesh,
        out_type=jax.ShapeDtypeStruct((B, NJG, S * 8), jnp.int32),
        scratch_types=[
            pltpu.VMEM((8, S), jnp.int32),
            pltpu.VMEM((S * 8,), jnp.int32),
        ],
        compiler_params=pltpu.CompilerParams(needs_layout_passes=False),
    )(_sc_scatter_body)(idx_t)
    pres4 = pres.reshape(B, NJG, S, 8)

    mask = pl.pallas_call(
        _mask_body,
        grid=(B, S // RBM),
        in_specs=[
            pl.BlockSpec((1, 1, RBM, 8),
                         functools.partial(lambda jg, b, ri: (b, jg, ri, 0), jg))
            for jg in range(NJG)
        ],
        out_specs=pl.BlockSpec((1, 1, RBM, S), lambda b, ri: (b, 0, ri, 0)),
        out_shape=jax.ShapeDtypeStruct((B, 1, S, S), jnp.bool_),
    )(*([pres4] * NJG))
    return mask


# speedup vs baseline: 1.0652x; 1.0652x over previous
import functools

import jax
import jax.numpy as jnp
from jax import lax
from jax.experimental import pallas as pl
from jax.experimental.pallas import tpu as pltpu
from jax.experimental.pallas import tpu_sc as plsc


B = 4
S = 2048
KK = 64
CB = 256
RBM = 256
NJG = 8
NC = 2
NSUB = 16


def _topk_body(scores_t_ref, idx_ref):
    vals = scores_t_ref[0]
    iota_c = lax.broadcasted_iota(jnp.int32, (S, CB), 0)
    iota_k = lax.broadcasted_iota(jnp.int32, (KK, CB), 0)
    acc0 = jnp.zeros((KK, CB), jnp.int32)

    def step(t, carry):
        vals, acc = carry
        v, ix = vals, iota_c
        while v.shape[0] > 1:
            h = v.shape[0] // 2
            a, b = v[:h], v[h:]
            ia, ib = ix[:h], ix[h:]
            gt = (a > b) | ((a == b) & (ia < ib))
            v = jnp.where(gt, a, b)
            ix = jnp.where(gt, ia, ib)
        am = ix
        acc = jnp.where(iota_k == t, am, acc)
        vals = jnp.where(iota_c == am, -jnp.inf, vals)
        return vals, acc

    _, acc = lax.fori_loop(0, KK, step, (vals, acc0))
    idx_ref[0] = acc


def _sc_scatter_body(idx_t_hbm, pres_hbm, idx_v, pres_v):
    wid = lax.axis_index("s") * NC + lax.axis_index("c")
    b = wid // NJG
    jg = wid % NJG
    pltpu.sync_copy(idx_t_hbm.at[b, pl.ds(jg * 8, 8)], idx_v)

    zeros = jnp.zeros((16,), jnp.int32)
    ones = jnp.ones((16,), jnp.int32)

    def zero_step(i, carry):
        pres_v[pl.ds(i * 16, 16)] = zeros
        return carry

    lax.fori_loop(0, S * 8 // 16, zero_step, 0)

    for jl in range(8):
        jl_vec = jnp.full((16,), jl, jnp.int32)

        def mark_step(i, carry):
            rows = idx_v[jl, pl.ds(i * 16, 16)]
            plsc.store_scatter(pres_v, [rows * 8 + jl_vec], ones)
            return carry

        lax.fori_loop(0, S // 16, mark_step, 0)

    pltpu.sync_copy(pres_v, pres_hbm.at[b, jg])


def _mask_body(p0, p1, p2, p3, p4, p5, p6, p7, out_ref):
    ri = pl.program_id(1)
    pres = jnp.concatenate(
        [p[0, 0] for p in (p0, p1, p2, p3, p4, p5, p6, p7)], axis=1)
    pres_full = jnp.concatenate(
        [pres, jnp.zeros((RBM, S - KK), dtype=jnp.int32)], axis=1)
    keep = pres_full == 0
    rr = lax.broadcasted_iota(jnp.int32, (RBM, S), 0) + ri * RBM
    jj = lax.broadcasted_iota(jnp.int32, (RBM, S), 1)
    override = (rr <= KK) & (jj > rr)
    out_ref[0, 0] = keep | override


def kernel(attn_scores):
    scores_t = jnp.swapaxes(attn_scores, 1, 2)

    idx_t = pl.pallas_call(
        _topk_body,
        grid=(B, S // CB),
        in_specs=[pl.BlockSpec((1, S, CB), lambda b, si: (b, 0, si))],
        out_specs=pl.BlockSpec((1, KK, CB), lambda b, si: (b, 0, si)),
        out_shape=jax.ShapeDtypeStruct((B, KK, S), jnp.int32),
    )(scores_t)

    mesh = plsc.VectorSubcoreMesh(core_axis_name="c", subcore_axis_name="s")
    pres = functools.partial(
        pl.kernel,
        mesh=mesh,
        out_type=jax.ShapeDtypeStruct((B, NJG, S * 8), jnp.int32),
        scratch_types=[
            pltpu.VMEM((8, S), jnp.int32),
            pltpu.VMEM((S * 8,), jnp.int32),
        ],
        compiler_params=pltpu.CompilerParams(needs_layout_passes=False),
    )(_sc_scatter_body)(idx_t)
    pres4 = pres.reshape(B, NJG, S, 8)

    mask = pl.pallas_call(
        _mask_body,
        grid=(B, S // RBM),
        in_specs=[
            pl.BlockSpec((1, 1, RBM, 8),
                         functools.partial(lambda jg, b, ri: (b, jg, ri, 0), jg))
            for jg in range(NJG)
        ],
        out_specs=pl.BlockSpec((1, 1, RBM, S), lambda b, ri: (b, 0, ri, 0)),
        out_shape=jax.ShapeDtypeStruct((B, 1, S, S), jnp.bool_),
    )(*([pres4] * NJG))
    return mask


# speedup vs baseline: 1.1045x; 1.0369x over previous
import functools

import jax
import jax.numpy as jnp
from jax import lax
from jax.experimental import pallas as pl
from jax.experimental.pallas import tpu as pltpu
from jax.experimental.pallas import tpu_sc as plsc


B = 4
S = 2048
KK = 64
CB = 128
RBM = 256
NJG = 8
NC = 2
NSUB = 16


def _topk_body(scores_t_ref, idx_ref):
    vals = scores_t_ref[0]
    iota_c = lax.broadcasted_iota(jnp.int32, (S, CB), 0)
    iota_k = lax.broadcasted_iota(jnp.int32, (KK, CB), 0)
    acc0 = jnp.zeros((KK, CB), jnp.int32)

    def step(t, carry):
        vals, acc = carry
        v, ix = vals, iota_c
        while v.shape[0] > 1:
            h = v.shape[0] // 2
            a, b = v[:h], v[h:]
            ia, ib = ix[:h], ix[h:]
            gt = (a > b) | ((a == b) & (ia < ib))
            v = jnp.where(gt, a, b)
            ix = jnp.where(gt, ia, ib)
        am = ix
        acc = jnp.where(iota_k == t, am, acc)
        vals = jnp.where(iota_c == am, -jnp.inf, vals)
        return vals, acc

    _, acc = lax.fori_loop(0, KK, step, (vals, acc0))
    idx_ref[0] = acc


def _sc_scatter_body(idx_t_hbm, pres_hbm, idx_v, pres_v):
    wid = lax.axis_index("s") * NC + lax.axis_index("c")
    b = wid // NJG
    jg = wid % NJG
    pltpu.sync_copy(idx_t_hbm.at[b, pl.ds(jg * 8, 8)], idx_v)

    zeros = jnp.zeros((16,), jnp.int32)
    ones = jnp.ones((16,), jnp.int32)

    def zero_step(i, carry):
        pres_v[pl.ds(i * 16, 16)] = zeros
        return carry

    lax.fori_loop(0, S * 8 // 16, zero_step, 0)

    for jl in range(8):
        jl_vec = jnp.full((16,), jl, jnp.int32)

        def mark_step(i, carry):
            rows = idx_v[jl, pl.ds(i * 16, 16)]
            plsc.store_scatter(pres_v, [rows * 8 + jl_vec], ones)
            return carry

        lax.fori_loop(0, S // 16, mark_step, 0)

    pltpu.sync_copy(pres_v, pres_hbm.at[b, jg])


def _mask_body(p0, p1, p2, p3, p4, p5, p6, p7, out_ref):
    ri = pl.program_id(1)
    pres = jnp.concatenate(
        [p[0, 0] for p in (p0, p1, p2, p3, p4, p5, p6, p7)], axis=1)
    pres_full = jnp.concatenate(
        [pres, jnp.zeros((RBM, S - KK), dtype=jnp.int32)], axis=1)
    keep = pres_full == 0
    rr = lax.broadcasted_iota(jnp.int32, (RBM, S), 0) + ri * RBM
    jj = lax.broadcasted_iota(jnp.int32, (RBM, S), 1)
    override = (rr <= KK) & (jj > rr)
    out_ref[0, 0] = keep | override


def kernel(attn_scores):
    scores_t = jnp.swapaxes(attn_scores, 1, 2)

    idx_t = pl.pallas_call(
        _topk_body,
        grid=(B, S // CB),
        in_specs=[pl.BlockSpec((1, S, CB), lambda b, si: (b, 0, si))],
        out_specs=pl.BlockSpec((1, KK, CB), lambda b, si: (b, 0, si)),
        out_shape=jax.ShapeDtypeStruct((B, KK, S), jnp.int32),
    )(scores_t)

    mesh = plsc.VectorSubcoreMesh(core_axis_name="c", subcore_axis_name="s")
    pres = functools.partial(
        pl.kernel,
        mesh=mesh,
        out_type=jax.ShapeDtypeStruct((B, NJG, S * 8), jnp.int32),
        scratch_types=[
            pltpu.VMEM((8, S), jnp.int32),
            pltpu.VMEM((S * 8,), jnp.int32),
        ],
        compiler_params=pltpu.CompilerParams(needs_layout_passes=False),
    )(_sc_scatter_body)(idx_t)
    pres4 = pres.reshape(B, NJG, S, 8)

    mask = pl.pallas_call(
        _mask_body,
        grid=(B, S // RBM),
        in_specs=[
            pl.BlockSpec((1, 1, RBM, 8),
                         functools.partial(lambda jg, b, ri: (b, jg, ri, 0), jg))
            for jg in range(NJG)
        ],
        out_specs=pl.BlockSpec((1, 1, RBM, S), lambda b, ri: (b, 0, ri, 0)),
        out_shape=jax.ShapeDtypeStruct((B, 1, S, S), jnp.bool_),
    )(*([pres4] * NJG))
    return mask
